# Initial kernel scaffold; baseline (speedup 1.0000x reference)
import jax
import jax.numpy as jnp
from jax.experimental import pallas as pl


def kernel(x, weight):
    raise NotImplementedError("write your pallas kernel here")


# speedup vs baseline: 3.5347x; 3.5347x over previous
import functools

import jax
import jax.numpy as jnp
from jax import lax
from jax.experimental import pallas as pl
from jax.experimental.pallas import tpu as pltpu
from jax.experimental.pallas import tpu_sc as plsc

_D = 64
_NC, _NS = 2, 16
_NW = _NC * _NS
_K = 128


@functools.cache
def _make(batch, hist, vocab):
    total = batch * hist
    assert total % (_NW * _K) == 0
    chunks = total // (_NW * _K)
    b_per_w = total // _NW
    mesh = plsc.VectorSubcoreMesh(core_axis_name="c", subcore_axis_name="s")

    @functools.partial(
        pl.kernel,
        out_type=jax.ShapeDtypeStruct((total, _D), jnp.float32),
        mesh=mesh,
        scratch_types=[
            pltpu.VMEM((chunks, _K), jnp.int32),
            pltpu.VMEM((_K, _D), jnp.float32),
            pltpu.SemaphoreType.DMA,
        ],
        compiler_params=pltpu.CompilerParams(use_tc_tiling_on_sc=False),
    )
    def emb(x_hbm, w_hbm, out_hbm, idx_v, rows_v, gsem):
        wid = lax.axis_index("s") * _NC + lax.axis_index("c")
        base = wid * b_per_w
        pltpu.sync_copy(x_hbm.at[wid], idx_v)

        def body(j, carry):
            pltpu.async_copy(w_hbm.at[idx_v.at[j]], rows_v, gsem).wait()
            pltpu.sync_copy(rows_v, out_hbm.at[pl.ds(base + j * _K, _K)])
            return carry

        lax.fori_loop(0, chunks, body, 0)

    return emb


def kernel(x, weight):
    batch, hist = x.shape
    vocab, d = weight.shape
    xr = x.reshape(_NW, (batch * hist) // (_NW * _K), _K)
    out = _make(batch, hist, vocab)(xr, weight)
    return out.reshape(batch, hist, d)


# speedup vs baseline: 4.2597x; 1.2051x over previous
import functools

import jax
import jax.numpy as jnp
from jax import lax
from jax.experimental import pallas as pl
from jax.experimental.pallas import tpu as pltpu
from jax.experimental.pallas import tpu_sc as plsc

_D = 64
_NC, _NS = 2, 16
_NW = _NC * _NS
_K = 128
_NBUF = 4


@functools.cache
def _make(batch, hist, vocab):
    total = batch * hist
    assert total % (_NW * _K) == 0
    chunks = total // (_NW * _K)
    b_per_w = total // _NW
    nrounds = chunks // _NBUF
    assert chunks % _NBUF == 0 and nrounds >= 2
    mesh = plsc.VectorSubcoreMesh(core_axis_name="c", subcore_axis_name="s")

    @functools.partial(
        pl.kernel,
        out_type=jax.ShapeDtypeStruct((total, _D), jnp.float32),
        mesh=mesh,
        scratch_types=[
            pltpu.VMEM((chunks, _K), jnp.int32),
            pltpu.VMEM((_NBUF, _K, _D), jnp.float32),
            [pltpu.SemaphoreType.DMA] * _NBUF,
            [pltpu.SemaphoreType.DMA] * _NBUF,
        ],
        compiler_params=pltpu.CompilerParams(use_tc_tiling_on_sc=False),
    )
    def emb(x_hbm, w_hbm, out_hbm, idx_v, rows_v, gsems, wsems):
        wid = lax.axis_index("s") * _NC + lax.axis_index("c")
        base = wid * b_per_w
        pltpu.sync_copy(x_hbm.at[wid], idx_v)

        def g_issue(c, b):
            pltpu.async_copy(w_hbm.at[idx_v.at[c]], rows_v.at[b], gsems[b])

        def g_wait(c, b):
            pltpu.make_async_copy(
                w_hbm.at[idx_v.at[c]], rows_v.at[b], gsems[b]).wait()

        def w_issue(c, b):
            pltpu.async_copy(
                rows_v.at[b], out_hbm.at[pl.ds(base + c * _K, _K)], wsems[b])

        def w_wait(c, b):
            pltpu.make_async_copy(
                rows_v.at[b], out_hbm.at[pl.ds(base + c * _K, _K)],
                wsems[b]).wait()

        for b in range(_NBUF - 1):
            g_issue(b, b)

        def do_round(c0, first, last):
            for b in range(_NBUF):
                c = c0 + b
                g_wait(c, b)
                w_issue(c, b)
                if last and b >= 1:
                    continue
                bp = (b + _NBUF - 1) % _NBUF
                if not (first and b == 0):
                    w_wait(c - 1, bp)
                g_issue(c + _NBUF - 1, bp)

        do_round(0, True, False)

        @pl.loop(_NBUF, (nrounds - 1) * _NBUF, step=_NBUF)
        def _rounds(c0):
            do_round(c0, False, False)

        do_round((nrounds - 1) * _NBUF, False, True)

        for b in range(_NBUF):
            w_wait((nrounds - 1) * _NBUF + b, b)

    return emb


def kernel(x, weight):
    batch, hist = x.shape
    vocab, d = weight.shape
    xr = x.reshape(_NW, (batch * hist) // (_NW * _K), _K)
    out = _make(batch, hist, vocab)(xr, weight)
    return out.reshape(batch, hist, d)
